# Initial kernel scaffold; baseline (speedup 1.0000x reference)
import jax
import jax.numpy as jnp
from jax.experimental import pallas as pl


def kernel(anchors, logits, bbox_regs, sizes):
    raise NotImplementedError("write your pallas kernel here")


# speedup vs baseline: 12.4124x; 12.4124x over previous
import functools
import math

import jax
import jax.numpy as jnp
from jax import lax
from jax.experimental import pallas as pl
from jax.experimental.pallas import tpu as pltpu

_PRE = 2000
_POST = 1000
_THR = 0.7
_P = 2048
_OUT_R = 1024
_BR = 128
_MAX_OFF = math.log(1000.0 / 16)


def _permute_nchw(t, C):
    N, _, H_, W_ = t.shape
    t = t.reshape(N, -1, C, H_, W_)
    t = jnp.transpose(t, (0, 3, 4, 1, 2))
    return t.reshape(N, -1, C)


def _rpn_body(anc_ref, reg_ref, s_ref, size_ref, out_ref, m_ref, tt_ref):
    f32 = jnp.float32
    anc = anc_ref[0]
    reg = reg_ref[0]
    s = s_ref[0]
    h_img = size_ref[0, 0, 0]
    w_img = size_ref[0, 0, 1]

    ax1, ay1, ax2, ay2 = anc[0:1], anc[1:2], anc[2:3], anc[3:4]
    ws = ax2 - ax1 + 1.0
    hs = ay2 - ay1 + 1.0
    xc = ax1 + 0.5 * ws
    yc = ay1 + 0.5 * hs
    dx, dy = reg[0:1], reg[1:2]
    dw = jnp.minimum(reg[2:3], _MAX_OFF)
    dh = jnp.minimum(reg[3:4], _MAX_OFF)
    xc = xc + dx * ws
    yc = yc + dy * hs
    ws = ws * jnp.exp(dw)
    hs = hs * jnp.exp(dh)
    x1 = xc - 0.5 * ws
    y1 = yc - 0.5 * hs
    x2 = xc + 0.5 * ws - 1.0
    y2 = yc + 0.5 * hs - 1.0
    x1 = jnp.minimum(jnp.maximum(x1, 0.0), w_img - 1.0)
    y1 = jnp.minimum(jnp.maximum(y1, 0.0), h_img - 1.0)
    x2 = jnp.minimum(jnp.maximum(x2, 0.0), w_img - 1.0)
    y2 = jnp.minimum(jnp.maximum(y2, 0.0), h_img - 1.0)
    area = (x2 - x1 + 1.0) * (y2 - y1 + 1.0)

    col_ids = lax.broadcasted_iota(jnp.int32, (1, _P), 1)
    valid = col_ids < _PRE

    t8 = jnp.concatenate(
        [x1, y1, x2, y2, s, area, jnp.zeros((2, _P), f32)], axis=0)
    tt_ref[...] = jnp.transpose(t8)

    for rb in range(_P // _BR):
        r0 = rb * _BR
        tb = tt_ref[r0:r0 + _BR, :]
        bx1, by1 = tb[:, 0:1], tb[:, 1:2]
        bx2, by2 = tb[:, 2:3], tb[:, 3:4]
        barea = tb[:, 5:6]
        x_tl = jnp.maximum(bx1, x1)
        y_tl = jnp.maximum(by1, y1)
        x_br = jnp.minimum(bx2, x2)
        y_br = jnp.minimum(by2, y2)
        inter = (jnp.maximum(x_br - x_tl + 1.0, 0.0)
                 * jnp.maximum(y_br - y_tl + 1.0, 0.0))
        iou = inter / (barea + area - inter)
        row_ids = r0 + lax.broadcasted_iota(jnp.int32, (_BR, 1), 0)
        m = ((iou > _THR) & (col_ids > row_ids) & valid
             & (row_ids < _PRE))
        m_ref[r0:r0 + _BR, :] = m.astype(jnp.bfloat16)

    keep0 = jnp.where(valid, 1.0, 0.0)

    def _cond(carry):
        return carry[1]

    def _body(carry):
        k, _ = carry
        sup = lax.dot_general(
            k.astype(jnp.bfloat16), m_ref[...],
            (((1,), (0,)), ((), ())), preferred_element_type=f32)
        k_new = jnp.where((sup > 0.0) | (~valid), 0.0, 1.0)
        changed = jnp.any(k_new != k)
        return (k_new, changed)

    keep, _ = lax.while_loop(_cond, _body, (keep0, jnp.bool_(True)))

    kv = keep
    nv = jnp.where(valid, 1.0 - keep, 0.0)

    def _cumsum_lanes(v):
        c = v
        sh = 1
        while sh < _P:
            c = c + jnp.concatenate(
                [jnp.zeros((1, sh), f32), c[:, :_P - sh]], axis=1)
            sh *= 2
        return c

    ckv = _cumsum_lanes(kv)
    cnv = _cumsum_lanes(nv)
    n_keep = ckv[:, _P - 1:_P]
    pos = jnp.where(keep > 0.0, ckv - 1.0, n_keep + cnv - 1.0)
    pos = jnp.where(valid, pos, 2.0 * _P)

    rows = lax.broadcasted_iota(jnp.int32, (_OUT_R, 1), 0).astype(f32)
    acc = jnp.zeros((_OUT_R, 8), f32)
    for c0 in range(0, _P, _OUT_R):
        oht = (rows == pos[:, c0:c0 + _OUT_R]).astype(f32)
        acc = acc + jnp.dot(oht, tt_ref[c0:c0 + _OUT_R, :],
                            preferred_element_type=f32)
    out_ref[0] = acc


@functools.partial(jax.jit, static_argnames=())
def kernel(anchors, logits, bbox_regs, sizes):
    N, A_, H_, W_ = logits.shape
    scores = _permute_nchw(logits, 1).reshape(N, -1)
    regs = _permute_nchw(bbox_regs, 4)
    scores = jax.nn.sigmoid(scores)
    s_k, topk_inds = lax.top_k(scores, _PRE)
    regs_k = jnp.take_along_axis(regs, topk_inds[:, :, None], axis=1)
    anc_k = jnp.take_along_axis(anchors, topk_inds[:, :, None], axis=1)

    pad = _P - _PRE
    anc_t = jnp.pad(jnp.transpose(anc_k, (0, 2, 1)), ((0, 0), (0, 0), (0, pad)))
    reg_t = jnp.pad(jnp.transpose(regs_k, (0, 2, 1)), ((0, 0), (0, 0), (0, pad)))
    s_p = jnp.pad(s_k, ((0, 0), (0, pad))).reshape(N, 1, _P)
    sizes3 = sizes.astype(jnp.float32).reshape(N, 1, 2)

    out = pl.pallas_call(
        _rpn_body,
        grid=(N,),
        in_specs=[
            pl.BlockSpec((1, 4, _P), lambda i: (i, 0, 0)),
            pl.BlockSpec((1, 4, _P), lambda i: (i, 0, 0)),
            pl.BlockSpec((1, 1, _P), lambda i: (i, 0, 0)),
            pl.BlockSpec((1, 1, 2), lambda i: (i, 0, 0)),
        ],
        out_specs=pl.BlockSpec((1, _OUT_R, 8), lambda i: (i, 0, 0)),
        out_shape=jax.ShapeDtypeStruct((N, _OUT_R, 8), jnp.float32),
        scratch_shapes=[
            pltpu.VMEM((_P, _P), jnp.bfloat16),
            pltpu.VMEM((_P, 8), jnp.float32),
        ],
    )(anc_t, reg_t, s_p, sizes3)

    boxes = out[:, :_POST, 0:4]
    out_scores = out[:, :_POST, 4]
    return boxes, out_scores


# speedup vs baseline: 14.9260x; 1.2025x over previous
import functools
import math

import jax
import jax.numpy as jnp
from jax import lax
from jax.experimental import pallas as pl
from jax.experimental.pallas import tpu as pltpu

_PRE = 2000
_POST = 1000
_THR = 0.7
_P = 2048
_OUT_R = 1024
_BR = 128
_MAX_OFF = math.log(1000.0 / 16)


def _permute_nchw(t, C):
    N, _, H_, W_ = t.shape
    t = t.reshape(N, -1, C, H_, W_)
    t = jnp.transpose(t, (0, 3, 4, 1, 2))
    return t.reshape(N, -1, C)


def _rpn_body(anc_ref, reg_ref, s_ref, size_ref, out_ref, m_ref, tt_ref):
    f32 = jnp.float32
    anc = anc_ref[0]
    reg = reg_ref[0]
    s = s_ref[0]
    h_img = size_ref[0, 0, 0]
    w_img = size_ref[0, 0, 1]

    ax1, ay1, ax2, ay2 = anc[0:1], anc[1:2], anc[2:3], anc[3:4]
    ws = ax2 - ax1 + 1.0
    hs = ay2 - ay1 + 1.0
    xc = ax1 + 0.5 * ws
    yc = ay1 + 0.5 * hs
    dx, dy = reg[0:1], reg[1:2]
    dw = jnp.minimum(reg[2:3], _MAX_OFF)
    dh = jnp.minimum(reg[3:4], _MAX_OFF)
    xc = xc + dx * ws
    yc = yc + dy * hs
    ws = ws * jnp.exp(dw)
    hs = hs * jnp.exp(dh)
    x1 = xc - 0.5 * ws
    y1 = yc - 0.5 * hs
    x2 = xc + 0.5 * ws - 1.0
    y2 = yc + 0.5 * hs - 1.0
    x1 = jnp.minimum(jnp.maximum(x1, 0.0), w_img - 1.0)
    y1 = jnp.minimum(jnp.maximum(y1, 0.0), h_img - 1.0)
    x2 = jnp.minimum(jnp.maximum(x2, 0.0), w_img - 1.0)
    y2 = jnp.minimum(jnp.maximum(y2, 0.0), h_img - 1.0)
    area = (x2 - x1 + 1.0) * (y2 - y1 + 1.0)

    col_ids = lax.broadcasted_iota(jnp.int32, (1, _P), 1)
    valid = col_ids < _PRE

    t8 = jnp.concatenate(
        [x1, y1, x2, y2, s, area, jnp.zeros((2, _P), f32)], axis=0)
    tt_ref[...] = jnp.transpose(t8)

    for rb in range(_P // _BR):
        r0 = rb * _BR
        tb = tt_ref[r0:r0 + _BR, :]
        bx1, by1 = tb[:, 0:1], tb[:, 1:2]
        bx2, by2 = tb[:, 2:3], tb[:, 3:4]
        barea = tb[:, 5:6]
        x_tl = jnp.maximum(bx1, x1)
        y_tl = jnp.maximum(by1, y1)
        x_br = jnp.minimum(bx2, x2)
        y_br = jnp.minimum(by2, y2)
        inter = (jnp.maximum(x_br - x_tl + 1.0, 0.0)
                 * jnp.maximum(y_br - y_tl + 1.0, 0.0))
        iou = inter / (barea + area - inter)
        row_ids = r0 + lax.broadcasted_iota(jnp.int32, (_BR, 1), 0)
        m = ((iou > _THR) & (col_ids > row_ids) & valid
             & (row_ids < _PRE))
        m_ref[r0:r0 + _BR, :] = m.astype(jnp.bfloat16)

    keep0 = jnp.where(valid, 1.0, 0.0)

    def _cond(carry):
        return carry[1]

    def _body(carry):
        k, _ = carry
        sup = lax.dot_general(
            k.astype(jnp.bfloat16), m_ref[...],
            (((1,), (0,)), ((), ())), preferred_element_type=f32)
        k_new = jnp.where((sup > 0.0) | (~valid), 0.0, 1.0)
        changed = jnp.any(k_new != k)
        return (k_new, changed)

    keep, _ = lax.while_loop(_cond, _body, (keep0, jnp.bool_(True)))

    kv = keep
    nv = jnp.where(valid, 1.0 - keep, 0.0)

    def _cumsum_lanes(v):
        c = v
        sh = 1
        while sh < _P:
            c = c + jnp.concatenate(
                [jnp.zeros((1, sh), f32), c[:, :_P - sh]], axis=1)
            sh *= 2
        return c

    ckv = _cumsum_lanes(kv)
    cnv = _cumsum_lanes(nv)
    n_keep = ckv[:, _P - 1:_P]
    pos = jnp.where(keep > 0.0, ckv - 1.0, n_keep + cnv - 1.0)
    pos = jnp.where(valid, pos, 2.0 * _P)

    rows = lax.broadcasted_iota(jnp.int32, (_OUT_R, 1), 0).astype(f32)
    acc = jnp.zeros((_OUT_R, 8), f32)
    for c0 in range(0, _P, _OUT_R):
        oht = (rows == pos[:, c0:c0 + _OUT_R]).astype(f32)
        acc = acc + jnp.dot(oht, tt_ref[c0:c0 + _OUT_R, :],
                            preferred_element_type=f32)
    out_ref[0] = acc


@functools.partial(jax.jit, static_argnames=())
def kernel(anchors, logits, bbox_regs, sizes):
    N, A_, H_, W_ = logits.shape
    scores = _permute_nchw(logits, 1).reshape(N, -1)
    scores = jax.nn.sigmoid(scores)
    s_k, topk_inds = lax.top_k(scores, _PRE)
    flat_regs = bbox_regs.reshape(N, A_ * 4, H_ * W_)
    a_idx = topk_inds % A_
    hw_idx = topk_inds // A_
    g = jnp.take_along_axis(flat_regs, hw_idx[:, None, :], axis=2)
    g = g.reshape(N, A_, 4, _PRE)
    regs_kt = jnp.take_along_axis(g, a_idx[:, None, None, :], axis=1)[:, 0]
    anc_k = jnp.take_along_axis(anchors, topk_inds[:, :, None], axis=1)

    pad = _P - _PRE
    anc_t = jnp.pad(jnp.transpose(anc_k, (0, 2, 1)), ((0, 0), (0, 0), (0, pad)))
    reg_t = jnp.pad(regs_kt, ((0, 0), (0, 0), (0, pad)))
    s_p = jnp.pad(s_k, ((0, 0), (0, pad))).reshape(N, 1, _P)
    sizes3 = sizes.astype(jnp.float32).reshape(N, 1, 2)

    out = pl.pallas_call(
        _rpn_body,
        grid=(N,),
        in_specs=[
            pl.BlockSpec((1, 4, _P), lambda i: (i, 0, 0)),
            pl.BlockSpec((1, 4, _P), lambda i: (i, 0, 0)),
            pl.BlockSpec((1, 1, _P), lambda i: (i, 0, 0)),
            pl.BlockSpec((1, 1, 2), lambda i: (i, 0, 0)),
        ],
        out_specs=pl.BlockSpec((1, _OUT_R, 8), lambda i: (i, 0, 0)),
        out_shape=jax.ShapeDtypeStruct((N, _OUT_R, 8), jnp.float32),
        scratch_shapes=[
            pltpu.VMEM((_P, _P), jnp.bfloat16),
            pltpu.VMEM((_P, 8), jnp.float32),
        ],
    )(anc_t, reg_t, s_p, sizes3)

    boxes = out[:, :_POST, 0:4]
    out_scores = out[:, :_POST, 4]
    return boxes, out_scores


# speedup vs baseline: 48.0095x; 3.2165x over previous
import functools
import math

import jax
import jax.numpy as jnp
from jax import lax
from jax.experimental import pallas as pl
from jax.experimental.pallas import tpu as pltpu

_PRE = 2000
_POST = 1000
_THR = 0.7
_P = 2048
_OUT_R = 1024
_BR = 128
_MAX_OFF = math.log(1000.0 / 16)


def _permute_nchw(t, C):
    N, _, H_, W_ = t.shape
    t = t.reshape(N, -1, C, H_, W_)
    t = jnp.transpose(t, (0, 3, 4, 1, 2))
    return t.reshape(N, -1, C)


def _rpn_body(anc_ref, reg_ref, s_ref, size_ref, out_ref, m_ref, tt_ref):
    f32 = jnp.float32
    anc = anc_ref[0]
    reg = reg_ref[0]
    s = s_ref[0]
    h_img = size_ref[0, 0, 0]
    w_img = size_ref[0, 0, 1]

    ax1, ay1, ax2, ay2 = anc[0:1], anc[1:2], anc[2:3], anc[3:4]
    ws = ax2 - ax1 + 1.0
    hs = ay2 - ay1 + 1.0
    xc = ax1 + 0.5 * ws
    yc = ay1 + 0.5 * hs
    dx, dy = reg[0:1], reg[1:2]
    dw = jnp.minimum(reg[2:3], _MAX_OFF)
    dh = jnp.minimum(reg[3:4], _MAX_OFF)
    xc = xc + dx * ws
    yc = yc + dy * hs
    ws = ws * jnp.exp(dw)
    hs = hs * jnp.exp(dh)
    x1 = xc - 0.5 * ws
    y1 = yc - 0.5 * hs
    x2 = xc + 0.5 * ws - 1.0
    y2 = yc + 0.5 * hs - 1.0
    x1 = jnp.minimum(jnp.maximum(x1, 0.0), w_img - 1.0)
    y1 = jnp.minimum(jnp.maximum(y1, 0.0), h_img - 1.0)
    x2 = jnp.minimum(jnp.maximum(x2, 0.0), w_img - 1.0)
    y2 = jnp.minimum(jnp.maximum(y2, 0.0), h_img - 1.0)
    area = (x2 - x1 + 1.0) * (y2 - y1 + 1.0)

    col_ids = lax.broadcasted_iota(jnp.int32, (1, _P), 1)
    valid = col_ids < _PRE

    t8 = jnp.concatenate(
        [x1, y1, x2, y2, s, area, jnp.zeros((2, _P), f32)], axis=0)
    tt_ref[...] = jnp.transpose(t8)

    for rb in range(_P // _BR):
        r0 = rb * _BR
        tb = tt_ref[r0:r0 + _BR, :]
        bx1, by1 = tb[:, 0:1], tb[:, 1:2]
        bx2, by2 = tb[:, 2:3], tb[:, 3:4]
        barea = tb[:, 5:6]
        x_tl = jnp.maximum(bx1, x1)
        y_tl = jnp.maximum(by1, y1)
        x_br = jnp.minimum(bx2, x2)
        y_br = jnp.minimum(by2, y2)
        inter = (jnp.maximum(x_br - x_tl + 1.0, 0.0)
                 * jnp.maximum(y_br - y_tl + 1.0, 0.0))
        iou = inter / (barea + area - inter)
        row_ids = r0 + lax.broadcasted_iota(jnp.int32, (_BR, 1), 0)
        m = ((iou > _THR) & (col_ids > row_ids) & valid
             & (row_ids < _PRE))
        m_ref[r0:r0 + _BR, :] = m.astype(jnp.bfloat16)

    keep0 = jnp.where(valid, 1.0, 0.0)

    def _cond(carry):
        return carry[1]

    def _body(carry):
        k, _ = carry
        sup = lax.dot_general(
            k.astype(jnp.bfloat16), m_ref[...],
            (((1,), (0,)), ((), ())), preferred_element_type=f32)
        k_new = jnp.where((sup > 0.0) | (~valid), 0.0, 1.0)
        changed = jnp.any(k_new != k)
        return (k_new, changed)

    keep, _ = lax.while_loop(_cond, _body, (keep0, jnp.bool_(True)))

    kv = keep
    nv = jnp.where(valid, 1.0 - keep, 0.0)

    def _cumsum_lanes(v):
        c = v
        sh = 1
        while sh < _P:
            c = c + jnp.concatenate(
                [jnp.zeros((1, sh), f32), c[:, :_P - sh]], axis=1)
            sh *= 2
        return c

    ckv = _cumsum_lanes(kv)
    cnv = _cumsum_lanes(nv)
    n_keep = ckv[:, _P - 1:_P]
    pos = jnp.where(keep > 0.0, ckv - 1.0, n_keep + cnv - 1.0)
    pos = jnp.where(valid, pos, 2.0 * _P)

    rows = lax.broadcasted_iota(jnp.int32, (_OUT_R, 1), 0).astype(f32)
    acc = jnp.zeros((_OUT_R, 8), f32)
    for c0 in range(0, _P, _OUT_R):
        oht = (rows == pos[:, c0:c0 + _OUT_R]).astype(f32)
        acc = acc + jnp.dot(oht, tt_ref[c0:c0 + _OUT_R, :],
                            preferred_element_type=f32)
    out_ref[0] = acc


@functools.partial(jax.jit, static_argnames=())
def kernel(anchors, logits, bbox_regs, sizes):
    N, A_, H_, W_ = logits.shape
    sl_scores = jax.nn.sigmoid(logits.reshape(N * A_, H_ * W_))
    v1, hw1 = lax.top_k(sl_scores, _PRE)
    i1 = (hw1 * A_
          + lax.broadcasted_iota(jnp.int32, (N * A_, _PRE), 0) % A_)
    sneg, si = lax.sort((-v1.reshape(N, A_ * _PRE),
                         i1.reshape(N, A_ * _PRE)), dimension=1, num_keys=2)
    s_k = -sneg[:, :_PRE]
    topk_inds = si[:, :_PRE]
    flat_regs = bbox_regs.reshape(N, A_ * 4, H_ * W_)
    a_idx = topk_inds % A_
    hw_idx = topk_inds // A_
    g = jnp.take_along_axis(flat_regs, hw_idx[:, None, :], axis=2)
    g = g.reshape(N, A_, 4, _PRE)
    regs_kt = jnp.take_along_axis(g, a_idx[:, None, None, :], axis=1)[:, 0]
    anc_k = jnp.take_along_axis(anchors, topk_inds[:, :, None], axis=1)

    pad = _P - _PRE
    anc_t = jnp.pad(jnp.transpose(anc_k, (0, 2, 1)), ((0, 0), (0, 0), (0, pad)))
    reg_t = jnp.pad(regs_kt, ((0, 0), (0, 0), (0, pad)))
    s_p = jnp.pad(s_k, ((0, 0), (0, pad))).reshape(N, 1, _P)
    sizes3 = sizes.astype(jnp.float32).reshape(N, 1, 2)

    out = pl.pallas_call(
        _rpn_body,
        grid=(N,),
        in_specs=[
            pl.BlockSpec((1, 4, _P), lambda i: (i, 0, 0)),
            pl.BlockSpec((1, 4, _P), lambda i: (i, 0, 0)),
            pl.BlockSpec((1, 1, _P), lambda i: (i, 0, 0)),
            pl.BlockSpec((1, 1, 2), lambda i: (i, 0, 0)),
        ],
        out_specs=pl.BlockSpec((1, _OUT_R, 8), lambda i: (i, 0, 0)),
        out_shape=jax.ShapeDtypeStruct((N, _OUT_R, 8), jnp.float32),
        scratch_shapes=[
            pltpu.VMEM((_P, _P), jnp.bfloat16),
            pltpu.VMEM((_P, 8), jnp.float32),
        ],
    )(anc_t, reg_t, s_p, sizes3)

    boxes = out[:, :_POST, 0:4]
    out_scores = out[:, :_POST, 4]
    return boxes, out_scores
